# Initial kernel scaffold; baseline (speedup 1.0000x reference)
import jax
import jax.numpy as jnp
from jax.experimental import pallas as pl


def kernel(xyz_samples, frame_index, table, W1, W2):
    raise NotImplementedError("write your pallas kernel here")


# speedup vs baseline: 332.2946x; 332.2946x over previous
import functools

import jax
import jax.numpy as jnp
import numpy as np
from jax import lax
from jax.experimental import pallas as pl
from jax.experimental.pallas import tpu as pltpu
from jax.experimental.pallas import tpu_sc as plsc

_L = 16
_T = 1024
_N = 262144
_B = float(np.exp(np.log(4096.0 / 16.0) / (_L - 1)))
_SCALES = [np.float32(16.0 * _B**l) for l in range(_L)]
_P2 = np.uint32(2654435761)
_P3 = np.uint32(805459861)
_MASK = np.uint32(_T - 1)

_NW = 32
_NPT = _N // _NW
_CHUNK = 2048
_NCH = _NPT // _CHUNK
_VECS = _CHUNK // 16
_NBLK = _NW * _NCH


def _sc_body(xs, ys, zs, t0h, t1h, outh, t0, t1, xv, yv, zv, fv):
    wid = lax.axis_index("s") * 2 + lax.axis_index("c")
    pltpu.sync_copy(t0h, t0)
    pltpu.sync_copy(t1h, t1)

    def chunk_body(ci, carry):
        blk = wid * _NCH + ci
        cbase = blk * _CHUNK
        pltpu.sync_copy(xs.at[pl.ds(cbase, _CHUNK)], xv)
        pltpu.sync_copy(ys.at[pl.ds(cbase, _CHUNK)], yv)
        pltpu.sync_copy(zs.at[pl.ds(cbase, _CHUNK)], zv)

        def vec_body(vi, carry2):
            o = vi * 16
            xn = (xv[pl.ds(o, 16)] + 1.0) * 0.5
            yn = (yv[pl.ds(o, 16)] + 1.0) * 0.5
            zn = (zv[pl.ds(o, 16)] + 1.0) * 0.5
            for l in range(_L):
                s = _SCALES[l]
                px = xn * s + 0.5
                py = yn * s + 0.5
                pz = zn * s + 0.5
                ix = px.astype(jnp.int32)
                iy = py.astype(jnp.int32)
                iz = pz.astype(jnp.int32)
                fx = px - ix.astype(jnp.float32)
                fy = py - iy.astype(jnp.float32)
                fz = pz - iz.astype(jnp.float32)
                a0 = plsc.bitcast(ix, jnp.uint32)
                a1 = a0 + jnp.uint32(1)
                b0 = plsc.bitcast(iy, jnp.uint32) * _P2
                b1 = b0 + _P2
                c0 = plsc.bitcast(iz, jnp.uint32) * _P3
                c1 = c0 + _P3
                off = jnp.uint32(l * _T)
                bc00 = b0 ^ c0
                bc01 = b0 ^ c1
                bc10 = b1 ^ c0
                bc11 = b1 ^ c1
                i000 = plsc.bitcast(((a0 ^ bc00) & _MASK) | off, jnp.int32)
                i001 = plsc.bitcast(((a0 ^ bc01) & _MASK) | off, jnp.int32)
                i010 = plsc.bitcast(((a0 ^ bc10) & _MASK) | off, jnp.int32)
                i011 = plsc.bitcast(((a0 ^ bc11) & _MASK) | off, jnp.int32)
                i100 = plsc.bitcast(((a1 ^ bc00) & _MASK) | off, jnp.int32)
                i101 = plsc.bitcast(((a1 ^ bc01) & _MASK) | off, jnp.int32)
                i110 = plsc.bitcast(((a1 ^ bc10) & _MASK) | off, jnp.int32)
                i111 = plsc.bitcast(((a1 ^ bc11) & _MASK) | off, jnp.int32)
                g000a = plsc.load_gather(t0, [i000])
                g001a = plsc.load_gather(t0, [i001])
                g010a = plsc.load_gather(t0, [i010])
                g011a = plsc.load_gather(t0, [i011])
                g100a = plsc.load_gather(t0, [i100])
                g101a = plsc.load_gather(t0, [i101])
                g110a = plsc.load_gather(t0, [i110])
                g111a = plsc.load_gather(t0, [i111])
                g000b = plsc.load_gather(t1, [i000])
                g001b = plsc.load_gather(t1, [i001])
                g010b = plsc.load_gather(t1, [i010])
                g011b = plsc.load_gather(t1, [i011])
                g100b = plsc.load_gather(t1, [i100])
                g101b = plsc.load_gather(t1, [i101])
                g110b = plsc.load_gather(t1, [i110])
                g111b = plsc.load_gather(t1, [i111])
                gx = 1.0 - fx
                gy = 1.0 - fy
                gz = 1.0 - fz
                w00 = gx * gy
                w01 = gx * fy
                w10 = fx * gy
                w11 = fx * fy
                w000 = w00 * gz
                w001 = w00 * fz
                w010 = w01 * gz
                w011 = w01 * fz
                w100 = w10 * gz
                w101 = w10 * fz
                w110 = w11 * gz
                w111 = w11 * fz
                f0 = (
                    (w000 * g000a + w001 * g001a)
                    + (w010 * g010a + w011 * g011a)
                ) + (
                    (w100 * g100a + w101 * g101a)
                    + (w110 * g110a + w111 * g111a)
                )
                f1 = (
                    (w000 * g000b + w001 * g001b)
                    + (w010 * g010b + w011 * g011b)
                ) + (
                    (w100 * g100b + w101 * g101b)
                    + (w110 * g110b + w111 * g111b)
                )
                fv[2 * l, pl.ds(o, 16)] = f0
                fv[2 * l + 1, pl.ds(o, 16)] = f1
            return carry2

        lax.fori_loop(0, _VECS, vec_body, 0)
        pltpu.sync_copy(fv, outh.at[blk])
        return carry

    lax.fori_loop(0, _NCH, chunk_body, 0)


@functools.cache
def _sc_encode():
    return pl.kernel(
        _sc_body,
        mesh=plsc.VectorSubcoreMesh(core_axis_name="c", subcore_axis_name="s"),
        compiler_params=pltpu.CompilerParams(needs_layout_passes=False),
        out_type=jax.ShapeDtypeStruct((_NBLK, 2 * _L, _CHUNK), jnp.float32),
        scratch_types=[
            pltpu.VMEM((_L * _T,), jnp.float32),
            pltpu.VMEM((_L * _T,), jnp.float32),
            pltpu.VMEM((_CHUNK,), jnp.float32),
            pltpu.VMEM((_CHUNK,), jnp.float32),
            pltpu.VMEM((_CHUNK,), jnp.float32),
            pltpu.VMEM((2 * _L, _CHUNK), jnp.float32),
        ],
    )


def _mlp_body(ft_ref, w1_ref, w2_ref, out_ref):
    ft = ft_ref[0]
    w1 = w1_ref[...]
    h = lax.dot_general(
        w1, ft, (((0,), (0,)), ((), ())), preferred_element_type=jnp.float32
    )
    sp = jnp.maximum(h, 0.0) + jnp.log1p(jnp.exp(-jnp.abs(h)))
    s = lax.dot_general(
        w2_ref[...], sp, (((1,), (0,)), ((), ())), preferred_element_type=jnp.float32
    )
    out_ref[0] = jnp.exp(s)


_mlp = pl.pallas_call(
    _mlp_body,
    grid=(_NBLK,),
    in_specs=[
        pl.BlockSpec((1, 2 * _L, _CHUNK), lambda i: (i, 0, 0)),
        pl.BlockSpec((2 * _L, 64), lambda i: (0, 0)),
        pl.BlockSpec((1, 64), lambda i: (0, 0)),
    ],
    out_specs=pl.BlockSpec((1, 1, _CHUNK), lambda i: (i, 0, 0)),
    out_shape=jax.ShapeDtypeStruct((_NBLK, 1, _CHUNK), jnp.float32),
)


def kernel(xyz_samples, frame_index, table, W1, W2):
    del frame_index
    xt = jnp.transpose(xyz_samples)
    t0 = table[:, :, 0].reshape(-1)
    t1 = table[:, :, 1].reshape(-1)
    feats = _sc_encode()(xt[0], xt[1], xt[2], t0, t1)
    w2row = W2[:, 0].reshape(1, 64)
    sig = _mlp(feats, W1, w2row)
    return sig.reshape(_N)


# speedup vs baseline: 353.9613x; 1.0652x over previous
import functools

import jax
import jax.numpy as jnp
import numpy as np
from jax import lax
from jax.experimental import pallas as pl
from jax.experimental.pallas import tpu as pltpu
from jax.experimental.pallas import tpu_sc as plsc

_L = 16
_T = 1024
_N = 262144
_B = float(np.exp(np.log(4096.0 / 16.0) / (_L - 1)))
_SCALES = [np.float32(16.0 * _B**l) for l in range(_L)]
_P2 = np.uint32(2654435761)
_P3 = np.uint32(805459861)
_MASK = np.uint32(_T - 1)

_NW = 32
_NPT = _N // _NW
_CHUNK = 2048
_NCH = _NPT // _CHUNK
_VECS = _CHUNK // 16
_NBLK = _NW * _NCH


def _sc_body(xs, ys, zs, t0h, t1h, outh, t0, t1, xv, yv, zv, fv):
    wid = lax.axis_index("s") * 2 + lax.axis_index("c")
    pltpu.sync_copy(t0h, t0)
    pltpu.sync_copy(t1h, t1)

    def chunk_body(ci, carry):
        blk = wid * _NCH + ci
        cbase = blk * _CHUNK
        pltpu.sync_copy(xs.at[pl.ds(cbase, _CHUNK)], xv)
        pltpu.sync_copy(ys.at[pl.ds(cbase, _CHUNK)], yv)
        pltpu.sync_copy(zs.at[pl.ds(cbase, _CHUNK)], zv)

        @plsc.parallel_loop(0, _VECS, 1, unroll=2)
        def vec_body(vi):
            o = vi * 16
            xn = (xv[pl.ds(o, 16)] + 1.0) * 0.5
            yn = (yv[pl.ds(o, 16)] + 1.0) * 0.5
            zn = (zv[pl.ds(o, 16)] + 1.0) * 0.5
            for l in range(_L):
                s = _SCALES[l]
                px = xn * s + 0.5
                py = yn * s + 0.5
                pz = zn * s + 0.5
                ix = px.astype(jnp.int32)
                iy = py.astype(jnp.int32)
                iz = pz.astype(jnp.int32)
                fx = px - ix.astype(jnp.float32)
                fy = py - iy.astype(jnp.float32)
                fz = pz - iz.astype(jnp.float32)
                a0 = plsc.bitcast(ix, jnp.uint32)
                a1 = a0 + jnp.uint32(1)
                b0 = plsc.bitcast(iy, jnp.uint32) * _P2
                b1 = b0 + _P2
                c0 = plsc.bitcast(iz, jnp.uint32) * _P3
                c1 = c0 + _P3
                bc00 = b0 ^ c0
                bc01 = b0 ^ c1
                bc10 = b1 ^ c0
                bc11 = b1 ^ c1
                i000 = plsc.bitcast((a0 ^ bc00) & _MASK, jnp.int32)
                i001 = plsc.bitcast((a0 ^ bc01) & _MASK, jnp.int32)
                i010 = plsc.bitcast((a0 ^ bc10) & _MASK, jnp.int32)
                i011 = plsc.bitcast((a0 ^ bc11) & _MASK, jnp.int32)
                i100 = plsc.bitcast((a1 ^ bc00) & _MASK, jnp.int32)
                i101 = plsc.bitcast((a1 ^ bc01) & _MASK, jnp.int32)
                i110 = plsc.bitcast((a1 ^ bc10) & _MASK, jnp.int32)
                i111 = plsc.bitcast((a1 ^ bc11) & _MASK, jnp.int32)
                t0l = t0.at[pl.ds(l * _T, _T)]
                t1l = t1.at[pl.ds(l * _T, _T)]
                g000a = plsc.load_gather(t0l, [i000])
                g001a = plsc.load_gather(t0l, [i001])
                g010a = plsc.load_gather(t0l, [i010])
                g011a = plsc.load_gather(t0l, [i011])
                g100a = plsc.load_gather(t0l, [i100])
                g101a = plsc.load_gather(t0l, [i101])
                g110a = plsc.load_gather(t0l, [i110])
                g111a = plsc.load_gather(t0l, [i111])
                g000b = plsc.load_gather(t1l, [i000])
                g001b = plsc.load_gather(t1l, [i001])
                g010b = plsc.load_gather(t1l, [i010])
                g011b = plsc.load_gather(t1l, [i011])
                g100b = plsc.load_gather(t1l, [i100])
                g101b = plsc.load_gather(t1l, [i101])
                g110b = plsc.load_gather(t1l, [i110])
                g111b = plsc.load_gather(t1l, [i111])
                gx = 1.0 - fx
                gy = 1.0 - fy
                gz = 1.0 - fz
                w00 = gx * gy
                w01 = gx * fy
                w10 = fx * gy
                w11 = fx * fy
                w000 = w00 * gz
                w001 = w00 * fz
                w010 = w01 * gz
                w011 = w01 * fz
                w100 = w10 * gz
                w101 = w10 * fz
                w110 = w11 * gz
                w111 = w11 * fz
                f0 = (
                    (w000 * g000a + w001 * g001a)
                    + (w010 * g010a + w011 * g011a)
                ) + (
                    (w100 * g100a + w101 * g101a)
                    + (w110 * g110a + w111 * g111a)
                )
                f1 = (
                    (w000 * g000b + w001 * g001b)
                    + (w010 * g010b + w011 * g011b)
                ) + (
                    (w100 * g100b + w101 * g101b)
                    + (w110 * g110b + w111 * g111b)
                )
                fv[2 * l, pl.ds(o, 16)] = f0
                fv[2 * l + 1, pl.ds(o, 16)] = f1

        pltpu.sync_copy(fv, outh.at[blk])
        return carry

    lax.fori_loop(0, _NCH, chunk_body, 0)


@functools.cache
def _sc_encode():
    return pl.kernel(
        _sc_body,
        mesh=plsc.VectorSubcoreMesh(core_axis_name="c", subcore_axis_name="s"),
        compiler_params=pltpu.CompilerParams(needs_layout_passes=False),
        out_type=jax.ShapeDtypeStruct((_NBLK, 2 * _L, _CHUNK), jnp.float32),
        scratch_types=[
            pltpu.VMEM((_L * _T,), jnp.float32),
            pltpu.VMEM((_L * _T,), jnp.float32),
            pltpu.VMEM((_CHUNK,), jnp.float32),
            pltpu.VMEM((_CHUNK,), jnp.float32),
            pltpu.VMEM((_CHUNK,), jnp.float32),
            pltpu.VMEM((2 * _L, _CHUNK), jnp.float32),
        ],
    )


def _mlp_body(ft_ref, w1_ref, w2_ref, out_ref):
    ft = ft_ref[0]
    w1 = w1_ref[...]
    h = lax.dot_general(
        w1, ft, (((0,), (0,)), ((), ())), preferred_element_type=jnp.float32
    )
    sp = jnp.maximum(h, 0.0) + jnp.log1p(jnp.exp(-jnp.abs(h)))
    s = lax.dot_general(
        w2_ref[...], sp, (((1,), (0,)), ((), ())), preferred_element_type=jnp.float32
    )
    out_ref[0] = jnp.exp(s)


_mlp = pl.pallas_call(
    _mlp_body,
    grid=(_NBLK,),
    in_specs=[
        pl.BlockSpec((1, 2 * _L, _CHUNK), lambda i: (i, 0, 0)),
        pl.BlockSpec((2 * _L, 64), lambda i: (0, 0)),
        pl.BlockSpec((1, 64), lambda i: (0, 0)),
    ],
    out_specs=pl.BlockSpec((1, 1, _CHUNK), lambda i: (i, 0, 0)),
    out_shape=jax.ShapeDtypeStruct((_NBLK, 1, _CHUNK), jnp.float32),
)


def kernel(xyz_samples, frame_index, table, W1, W2):
    del frame_index
    xt = jnp.transpose(xyz_samples)
    t0 = table[:, :, 0].reshape(-1)
    t1 = table[:, :, 1].reshape(-1)
    feats = _sc_encode()(xt[0], xt[1], xt[2], t0, t1)
    w2row = W2[:, 0].reshape(1, 64)
    sig = _mlp(feats, W1, w2row)
    return sig.reshape(_N)
